# Initial kernel scaffold; baseline (speedup 1.0000x reference)
import jax
import jax.numpy as jnp
from jax.experimental import pallas as pl


def kernel(x, edge_index, W1, att_src1, att_dst1, W2, att_src2, att_dst2):
    raise NotImplementedError("write your pallas kernel here")


# speedup vs baseline: 5.7547x; 5.7547x over previous
import functools

import jax
import jax.numpy as jnp
from jax.experimental import pallas as pl
from jax.experimental.pallas import tpu as pltpu

N = 10000
E = 320000
IN_CH = 128
HID = 16
HEADS = 8
OUT_CH = 64

ROWS = 1000


def _proj_kernel(x_ref, w_ref, as_ref, ad_ref, h_ref, a_ref, mx_ref):
    h = jnp.dot(x_ref[...], w_ref[...], preferred_element_type=jnp.float32)
    h_ref[...] = h
    a_src = jnp.dot(h, as_ref[...], preferred_element_type=jnp.float32)
    a_dst = jnp.dot(h, ad_ref[...], preferred_element_type=jnp.float32)
    nh = a_src.shape[-1]
    a_ref[...] = jnp.concatenate([a_src, a_dst], axis=-1)
    mx = jnp.concatenate(
        [jnp.max(a_src, axis=0, keepdims=True),
         jnp.max(a_dst, axis=0, keepdims=True)], axis=-1)
    mx_ref[...] = jnp.broadcast_to(mx[None], mx_ref.shape)


def _project(x, W, As, Ad):
    F = W.shape[1]
    H2 = 2 * As.shape[1]
    grid = N // ROWS
    return pl.pallas_call(
        _proj_kernel,
        grid=(grid,),
        in_specs=[
            pl.BlockSpec((ROWS, x.shape[1]), lambda i: (i, 0)),
            pl.BlockSpec((x.shape[1], F), lambda i: (0, 0)),
            pl.BlockSpec((F, As.shape[1]), lambda i: (0, 0)),
            pl.BlockSpec((F, As.shape[1]), lambda i: (0, 0)),
        ],
        out_specs=[
            pl.BlockSpec((ROWS, F), lambda i: (i, 0)),
            pl.BlockSpec((ROWS, H2), lambda i: (i, 0)),
            pl.BlockSpec((1, 8, H2), lambda i: (i, 0, 0)),
        ],
        out_shape=[
            jax.ShapeDtypeStruct((N, F), jnp.float32),
            jax.ShapeDtypeStruct((N, H2), jnp.float32),
            jax.ShapeDtypeStruct((grid, 8, H2), jnp.float32),
        ],
    )(x, W, As, Ad)


def _norm_elu_proj_kernel(acc_ref, den_ref, w_ref, as_ref, ad_ref,
                          h_ref, a_ref, mx_ref):
    den = den_ref[...]
    r, hh = den.shape
    denb = jnp.repeat(den[:, :, None], HID, axis=2).reshape(r, hh * HID)
    v = acc_ref[...] / (denb + 1e-16)
    v = jnp.where(v > 0, v, jnp.exp(v) - 1.0)
    h = jnp.dot(v, w_ref[...], preferred_element_type=jnp.float32)
    h_ref[...] = h
    a_src = jnp.dot(h, as_ref[...], preferred_element_type=jnp.float32)
    a_dst = jnp.dot(h, ad_ref[...], preferred_element_type=jnp.float32)
    a_ref[...] = jnp.concatenate([a_src, a_dst], axis=-1)
    mx = jnp.concatenate(
        [jnp.max(a_src, axis=0, keepdims=True),
         jnp.max(a_dst, axis=0, keepdims=True)], axis=-1)
    mx_ref[...] = jnp.broadcast_to(mx[None], mx_ref.shape)


def _norm_elu_project(acc, den, W, As, Ad):
    F = W.shape[1]
    H2 = 2 * As.shape[1]
    grid = N // ROWS
    return pl.pallas_call(
        _norm_elu_proj_kernel,
        grid=(grid,),
        in_specs=[
            pl.BlockSpec((ROWS, acc.shape[1]), lambda i: (i, 0)),
            pl.BlockSpec((ROWS, den.shape[1]), lambda i: (i, 0)),
            pl.BlockSpec((acc.shape[1], F), lambda i: (0, 0)),
            pl.BlockSpec((F, As.shape[1]), lambda i: (0, 0)),
            pl.BlockSpec((F, As.shape[1]), lambda i: (0, 0)),
        ],
        out_specs=[
            pl.BlockSpec((ROWS, F), lambda i: (i, 0)),
            pl.BlockSpec((ROWS, H2), lambda i: (i, 0)),
            pl.BlockSpec((1, 8, H2), lambda i: (i, 0, 0)),
        ],
        out_shape=[
            jax.ShapeDtypeStruct((N, F), jnp.float32),
            jax.ShapeDtypeStruct((N, H2), jnp.float32),
            jax.ShapeDtypeStruct((grid, 8, H2), jnp.float32),
        ],
    )(acc, den, W, As, Ad)


def _final_norm_kernel(acc_ref, den_ref, out_ref):
    out_ref[...] = acc_ref[...] / (den_ref[...] + 1e-16)


def _final_norm(acc, den):
    grid = N // ROWS
    return pl.pallas_call(
        _final_norm_kernel,
        grid=(grid,),
        in_specs=[
            pl.BlockSpec((ROWS, OUT_CH), lambda i: (i, 0)),
            pl.BlockSpec((ROWS, OUT_CH), lambda i: (i, 0)),
        ],
        out_specs=pl.BlockSpec((ROWS, OUT_CH), lambda i: (i, 0)),
        out_shape=jax.ShapeDtypeStruct((N, OUT_CH), jnp.float32),
    )(acc, den)


def _edge_pass_xla(h, a, shift, src, dst, heads, ch):
    a_src = a[:, :heads]
    a_dst = a[:, heads:]
    alpha = a_src[src] + a_dst[dst]
    alpha = jnp.where(alpha >= 0, alpha, 0.2 * alpha)
    ex = jnp.exp(alpha - shift[None, :])
    den = jax.ops.segment_sum(ex, dst, num_segments=N)
    msg = h[src].reshape(E, heads, ch) * ex[..., None]
    acc = jax.ops.segment_sum(msg.reshape(E, heads * ch), dst, num_segments=N)
    return acc, den


def kernel(x, edge_index, W1, att_src1, att_dst1, W2, att_src2, att_dst2):
    src = edge_index[0]
    dst = edge_index[1]
    eye = jnp.eye(HEADS, dtype=jnp.float32)
    As1 = (att_src1[0][:, :, None] * eye[:, None, :]).reshape(HEADS * HID, HEADS)
    Ad1 = (att_dst1[0][:, :, None] * eye[:, None, :]).reshape(HEADS * HID, HEADS)
    As2 = att_src2[0, 0][:, None]
    Ad2 = att_dst2[0, 0][:, None]

    h1, a1, mx1 = _project(x, W1, As1, Ad1)
    bound1 = (jnp.max(mx1[:, 0, :HEADS], axis=0)
              + jnp.max(mx1[:, 0, HEADS:], axis=0))
    shift1 = jnp.where(bound1 >= 0, bound1, 0.2 * bound1)

    acc1, den1 = _edge_pass_xla(h1, a1, shift1, src, dst, HEADS, HID)

    h2, a2, mx2 = _norm_elu_project(acc1, den1, W2, As2, Ad2)
    bound2 = jnp.max(mx2[:, 0, :1], axis=0) + jnp.max(mx2[:, 0, 1:], axis=0)
    shift2 = jnp.where(bound2 >= 0, bound2, 0.2 * bound2)

    acc2, den2 = _edge_pass_xla(h2, a2, shift2, src, dst, 1, OUT_CH)
    den2b = jnp.repeat(den2, OUT_CH, axis=1)
    return _final_norm(acc2, den2b)


# speedup vs baseline: 31.1515x; 5.4133x over previous
import functools

import jax
import jax.numpy as jnp
from jax import lax
from jax.experimental import pallas as pl
from jax.experimental.pallas import tpu as pltpu
from jax.experimental.pallas import tpu_sc as plsc

N = 10000
E = 320000
IN_CH = 128
HID = 16
HEADS = 8
OUT_CH = 64

ROWS = 1000

NC = 2
NS = 16
NW = NC * NS
B = 128
CH = 80
PW = B * CH
EPAD = NW * PW
NPAD = 10112
RPT = NPAD // NS
PADSHIFT = 40.0


def _proj_kernel(x_ref, w_ref, as_ref, ad_ref, tab_ref, adst_ref, mx_ref):
    h = jnp.dot(x_ref[...], w_ref[...], preferred_element_type=jnp.float32)
    a_src = jnp.dot(h, as_ref[...], preferred_element_type=jnp.float32)
    a_dst = jnp.dot(h, ad_ref[...], preferred_element_type=jnp.float32)
    r = h.shape[0]
    nh = a_src.shape[-1]
    z = jnp.zeros((r, 16 - nh), jnp.float32)
    tab_ref[...] = jnp.concatenate([h, a_src, z], axis=-1)
    adst_ref[...] = jnp.concatenate([a_dst, z], axis=-1)
    mx = jnp.concatenate(
        [jnp.max(a_src, axis=0, keepdims=True),
         jnp.max(a_dst, axis=0, keepdims=True)], axis=-1)
    mx_ref[...] = jnp.broadcast_to(mx[None], mx_ref.shape)


def _project(x, W, As, Ad):
    c = W.shape[1]
    nh = As.shape[1]
    rt = c + 16
    grid = N // ROWS
    return pl.pallas_call(
        _proj_kernel,
        grid=(grid,),
        in_specs=[
            pl.BlockSpec((ROWS, x.shape[1]), lambda i: (i, 0)),
            pl.BlockSpec((x.shape[1], c), lambda i: (0, 0)),
            pl.BlockSpec((c, nh), lambda i: (0, 0)),
            pl.BlockSpec((c, nh), lambda i: (0, 0)),
        ],
        out_specs=[
            pl.BlockSpec((ROWS, rt), lambda i: (i, 0)),
            pl.BlockSpec((ROWS, 16), lambda i: (i, 0)),
            pl.BlockSpec((1, 8, 2 * nh), lambda i: (i, 0, 0)),
        ],
        out_shape=[
            jax.ShapeDtypeStruct((N, rt), jnp.float32),
            jax.ShapeDtypeStruct((N, 16), jnp.float32),
            jax.ShapeDtypeStruct((grid, 8, 2 * nh), jnp.float32),
        ],
    )(x, W, As, Ad)


def _combine_proj_kernel(c_in, h_in, parts_ref, w_ref, as_ref, ad_ref,
                         tab_ref, adst_ref, mx_ref):
    p = parts_ref[...]
    s = p[0] + p[1]
    acc = s[:, :c_in]
    den = s[:, c_in:c_in + h_in]
    r = acc.shape[0]
    per = c_in // h_in
    denb = jnp.repeat(den[:, :, None], per, axis=2).reshape(r, c_in)
    v = acc / (denb + 1e-16)
    v = jnp.where(v > 0, v, jnp.exp(v) - 1.0)
    h = jnp.dot(v, w_ref[...], preferred_element_type=jnp.float32)
    a_src = jnp.dot(h, as_ref[...], preferred_element_type=jnp.float32)
    a_dst = jnp.dot(h, ad_ref[...], preferred_element_type=jnp.float32)
    nh = a_src.shape[-1]
    z = jnp.zeros((r, 16 - nh), jnp.float32)
    tab_ref[...] = jnp.concatenate([h, a_src, z], axis=-1)
    adst_ref[...] = jnp.concatenate([a_dst, z], axis=-1)
    mx = jnp.concatenate(
        [jnp.max(a_src, axis=0, keepdims=True),
         jnp.max(a_dst, axis=0, keepdims=True)], axis=-1)
    mx_ref[...] = jnp.broadcast_to(mx[None], mx_ref.shape)


def _combine_project(parts, W, As, Ad, c_in, h_in):
    rt_in = parts.shape[-1]
    co = W.shape[1]
    ho = As.shape[1]
    rt = co + 16
    grid = N // ROWS
    return pl.pallas_call(
        functools.partial(_combine_proj_kernel, c_in, h_in),
        grid=(grid,),
        in_specs=[
            pl.BlockSpec((2, ROWS, rt_in), lambda i: (0, i, 0)),
            pl.BlockSpec((c_in, co), lambda i: (0, 0)),
            pl.BlockSpec((co, ho), lambda i: (0, 0)),
            pl.BlockSpec((co, ho), lambda i: (0, 0)),
        ],
        out_specs=[
            pl.BlockSpec((ROWS, rt), lambda i: (i, 0)),
            pl.BlockSpec((ROWS, 16), lambda i: (i, 0)),
            pl.BlockSpec((1, 8, 2 * ho), lambda i: (i, 0, 0)),
        ],
        out_shape=[
            jax.ShapeDtypeStruct((N, rt), jnp.float32),
            jax.ShapeDtypeStruct((N, 16), jnp.float32),
            jax.ShapeDtypeStruct((grid, 8, 2 * ho), jnp.float32),
        ],
    )(parts, W, As, Ad)


def _final_kernel(parts_ref, out_ref):
    p = parts_ref[...]
    s = p[0] + p[1]
    acc = s[:, :OUT_CH]
    den = s[:, OUT_CH:OUT_CH + 1]
    out_ref[...] = acc / (den + 1e-16)


def _final_norm(parts):
    rt_in = parts.shape[-1]
    grid = N // ROWS
    return pl.pallas_call(
        _final_kernel,
        grid=(grid,),
        in_specs=[pl.BlockSpec((2, ROWS, rt_in), lambda i: (0, i, 0))],
        out_specs=pl.BlockSpec((ROWS, OUT_CH), lambda i: (i, 0)),
        out_shape=jax.ShapeDtypeStruct((N, OUT_CH), jnp.float32),
    )(parts)


_GDN = jax.lax.GatherDimensionNumbers(
    offset_dims=(), collapsed_slice_dims=(0,), start_index_map=(0,))


def _bcast_lane(v, lane):
    idx = jnp.full((16,), lane, jnp.int32)
    return jax.lax.gather(
        v, idx[:, None], _GDN, (1,),
        mode=jax.lax.GatherScatterMode.PROMISE_IN_BOUNDS)


def _edge_pass_sc(table, adstp, shift16, srcp, dstp, zacc, c, h_heads):
    rt = table.shape[-1]
    ngrp = c // 16
    per = c // h_heads
    mesh = plsc.VectorSubcoreMesh(core_axis_name="c", subcore_axis_name="s",
                                  num_cores=NC, num_subcores=NS)

    def body(tab_hbm, adst_hbm, sh_hbm, src_hbm, dst_hbm, zacc_hbm,
             parts_hbm, shv, srcv, dstv, trows, adr, acc_sh,
             s_tab, s_ad, s_sc):
        cid = lax.axis_index("c")
        sid = lax.axis_index("s")
        wid = sid * NC + cid

        @pl.when(sid == 0)
        def _():
            pltpu.sync_copy(zacc_hbm, acc_sh)
        pltpu.sync_copy(sh_hbm, shv)
        plsc.subcore_barrier()
        shvec = shv[...]

        def edge(e, carry):
            va = trows[e, pl.ds(c, 16)]
            vb = adr[e, :]
            al = va + vb
            al = jnp.where(al >= 0, al, 0.2 * al) - shvec
            ex = jnp.exp(al)
            trows[e, pl.ds(c, 16)] = ex
            for j in range(ngrp):
                bc = _bcast_lane(ex, (16 * j) // per)
                trows[e, pl.ds(16 * j, 16)] = trows[e, pl.ds(16 * j, 16)] * bc
            return carry

        def chunk(k, carry):
            base = wid * PW + k * B
            pltpu.sync_copy(src_hbm.at[pl.ds(base, B)], srcv)
            pltpu.sync_copy(dst_hbm.at[pl.ds(base, B)], dstv)
            pltpu.async_copy(tab_hbm.at[srcv], trows, s_tab).wait()
            pltpu.async_copy(adst_hbm.at[dstv], adr, s_ad).wait()
            lax.fori_loop(0, B, edge, 0, unroll=4)
            pltpu.async_copy(trows, acc_sh.at[dstv], s_sc, add=True).wait()
            return carry

        lax.fori_loop(0, CH, chunk, 0)
        plsc.subcore_barrier()
        pltpu.sync_copy(acc_sh.at[pl.ds(sid * RPT, RPT)],
                        parts_hbm.at[cid, pl.ds(sid * RPT, RPT)])

    f = pl.kernel(
        body,
        out_type=jax.ShapeDtypeStruct((NC, NPAD, rt), jnp.float32),
        mesh=mesh,
        compiler_params=pltpu.CompilerParams(use_tc_tiling_on_sc=False),
        scratch_types=[
            pltpu.VMEM((16,), jnp.float32),
            pltpu.VMEM((B,), jnp.int32),
            pltpu.VMEM((B,), jnp.int32),
            pltpu.VMEM((B, rt), jnp.float32),
            pltpu.VMEM((B, 16), jnp.float32),
            pltpu.VMEM_SHARED((NPAD, rt), jnp.float32),
            pltpu.SemaphoreType.DMA,
            pltpu.SemaphoreType.DMA,
            pltpu.SemaphoreType.DMA,
        ],
    )
    return f(table, adstp, shift16, srcp, dstp, zacc)


def kernel(x, edge_index, W1, att_src1, att_dst1, W2, att_src2, att_dst2):
    src = edge_index[0]
    dst = edge_index[1]
    npad_e = EPAD - E
    srcp = jnp.concatenate([src, jnp.zeros((npad_e,), jnp.int32)])
    dstp = jnp.concatenate([dst, jnp.full((npad_e,), N, jnp.int32)])

    eye = jnp.eye(HEADS, dtype=jnp.float32)
    As1 = (att_src1[0][:, :, None] * eye[:, None, :]).reshape(HEADS * HID, HEADS)
    Ad1 = (att_dst1[0][:, :, None] * eye[:, None, :]).reshape(HEADS * HID, HEADS)
    As2 = att_src2[0, 0][:, None]
    Ad2 = att_dst2[0, 0][:, None]

    c1 = HEADS * HID

    tab1, adst1, mx1 = _project(x, W1, As1, Ad1)
    bound1 = (jnp.max(mx1[:, 0, :HEADS], axis=0)
              + jnp.max(mx1[:, 0, HEADS:], axis=0))
    shift1 = jnp.where(bound1 >= 0, bound1, 0.2 * bound1)
    sh1 = jnp.concatenate(
        [shift1, jnp.full((16 - HEADS,), PADSHIFT, jnp.float32)])
    adst1p = jnp.pad(adst1, ((0, NPAD - N), (0, 0)))

    zacc1 = jnp.zeros((NPAD, c1 + 16), jnp.float32)
    parts1 = _edge_pass_sc(tab1, adst1p, sh1, srcp, dstp, zacc1, c1, HEADS)

    tab2, adst2, mx2 = _combine_project(parts1, W2, As2, Ad2, c1, HEADS)
    bound2 = jnp.max(mx2[:, 0, :1], axis=0) + jnp.max(mx2[:, 0, 1:], axis=0)
    shift2 = jnp.where(bound2 >= 0, bound2, 0.2 * bound2)
    sh2 = jnp.concatenate(
        [shift2, jnp.full((15,), PADSHIFT, jnp.float32)])
    adst2p = jnp.pad(adst2, ((0, NPAD - N), (0, 0)))

    zacc2 = jnp.zeros((NPAD, OUT_CH + 16), jnp.float32)
    parts2 = _edge_pass_sc(tab2, adst2p, sh2, srcp, dstp, zacc2, OUT_CH, 1)

    return _final_norm(parts2)


# speedup vs baseline: 35.3319x; 1.1342x over previous
import functools

import jax
import jax.numpy as jnp
from jax import lax
from jax.experimental import pallas as pl
from jax.experimental.pallas import tpu as pltpu
from jax.experimental.pallas import tpu_sc as plsc

N = 10000
E = 320000
IN_CH = 128
HID = 16
HEADS = 8
OUT_CH = 64

ROWS = 1000

NC = 2
NS = 16
NW = NC * NS
B = 128
EPAD = 327680
NPAD = 10112
RPT = NPAD // NS
PADSHIFT = 40.0
RT = 80


def _proj_kernel(x_ref, w_ref, as_ref, ad_ref,
                 tlo_ref, alo_ref, mx_ref):
    h = jnp.dot(x_ref[...], w_ref[...], preferred_element_type=jnp.float32)
    a_src = jnp.dot(h, as_ref[...], preferred_element_type=jnp.float32)
    a_dst = jnp.dot(h, ad_ref[...], preferred_element_type=jnp.float32)
    r = h.shape[0]
    hh = HEADS // 2
    z = jnp.zeros((r, 16 - hh), jnp.float32)
    tlo = jnp.concatenate([h[:, :64], z, a_src[:, :hh]], axis=-1)
    thi = jnp.concatenate([h[:, 64:], z, a_src[:, hh:]], axis=-1)
    alo = jnp.concatenate([z, a_dst[:, :hh]], axis=-1)
    ahi = jnp.concatenate([z, a_dst[:, hh:]], axis=-1)
    tlo_ref[...] = jnp.stack([tlo, thi])
    alo_ref[...] = jnp.stack([alo, ahi])
    mx = jnp.concatenate(
        [jnp.max(a_src, axis=0, keepdims=True),
         jnp.max(a_dst, axis=0, keepdims=True)], axis=-1)
    mx_ref[...] = jnp.broadcast_to(mx[None], mx_ref.shape)


def _project(x, W, As, Ad):
    c = W.shape[1]
    nh = As.shape[1]
    grid = N // ROWS
    return pl.pallas_call(
        _proj_kernel,
        grid=(grid,),
        in_specs=[
            pl.BlockSpec((ROWS, x.shape[1]), lambda i: (i, 0)),
            pl.BlockSpec((x.shape[1], c), lambda i: (0, 0)),
            pl.BlockSpec((c, nh), lambda i: (0, 0)),
            pl.BlockSpec((c, nh), lambda i: (0, 0)),
        ],
        out_specs=[
            pl.BlockSpec((2, ROWS, RT), lambda i: (0, i, 0)),
            pl.BlockSpec((2, ROWS, 16), lambda i: (0, i, 0)),
            pl.BlockSpec((1, 8, 2 * nh), lambda i: (i, 0, 0)),
        ],
        out_shape=[
            jax.ShapeDtypeStruct((2, N, RT), jnp.float32),
            jax.ShapeDtypeStruct((2, N, 16), jnp.float32),
            jax.ShapeDtypeStruct((grid, 8, 2 * nh), jnp.float32),
        ],
    )(x, W, As, Ad)


def _combine_proj_kernel(parts_ref, w_ref, as_ref, ad_ref,
                         tab_ref, adst_ref, mx_ref):
    p = parts_ref[...]
    acc = jnp.concatenate([p[0, :, :64], p[1, :, :64]], axis=-1)
    den = jnp.concatenate([p[0, :, 76:80], p[1, :, 76:80]], axis=-1)
    r = acc.shape[0]
    denb = jnp.repeat(den[:, :, None], HID, axis=2).reshape(r, HEADS * HID)
    v = acc / (denb + 1e-16)
    v = jnp.where(v > 0, v, jnp.exp(v) - 1.0)
    h = jnp.dot(v, w_ref[...], preferred_element_type=jnp.float32)
    a_src = jnp.dot(h, as_ref[...], preferred_element_type=jnp.float32)
    a_dst = jnp.dot(h, ad_ref[...], preferred_element_type=jnp.float32)
    z = jnp.zeros((r, 15), jnp.float32)
    tab_ref[...] = jnp.concatenate([h, z, a_src], axis=-1)
    adst_ref[...] = jnp.concatenate([z, a_dst], axis=-1)
    mx = jnp.concatenate(
        [jnp.max(a_src, axis=0, keepdims=True),
         jnp.max(a_dst, axis=0, keepdims=True)], axis=-1)
    mx_ref[...] = jnp.broadcast_to(mx[None], mx_ref.shape)


def _combine_project(parts, W, As, Ad):
    co = W.shape[1]
    ho = As.shape[1]
    grid = N // ROWS
    return pl.pallas_call(
        _combine_proj_kernel,
        grid=(grid,),
        in_specs=[
            pl.BlockSpec((2, ROWS, RT), lambda i: (0, i, 0)),
            pl.BlockSpec((HEADS * HID, co), lambda i: (0, 0)),
            pl.BlockSpec((co, ho), lambda i: (0, 0)),
            pl.BlockSpec((co, ho), lambda i: (0, 0)),
        ],
        out_specs=[
            pl.BlockSpec((ROWS, RT), lambda i: (i, 0)),
            pl.BlockSpec((ROWS, 16), lambda i: (i, 0)),
            pl.BlockSpec((1, 8, 2 * ho), lambda i: (i, 0, 0)),
        ],
        out_shape=[
            jax.ShapeDtypeStruct((N, RT), jnp.float32),
            jax.ShapeDtypeStruct((N, 16), jnp.float32),
            jax.ShapeDtypeStruct((grid, 8, 2 * ho), jnp.float32),
        ],
    )(parts, W, As, Ad)


def _final_kernel(parts_ref, out_ref):
    p = parts_ref[...]
    s = p[0] + p[1]
    acc = s[:, :OUT_CH]
    den = s[:, 79:80]
    out_ref[...] = acc / (den + 1e-16)


def _final_norm(parts):
    grid = N // ROWS
    return pl.pallas_call(
        _final_kernel,
        grid=(grid,),
        in_specs=[pl.BlockSpec((2, ROWS, RT), lambda i: (0, i, 0))],
        out_specs=pl.BlockSpec((ROWS, OUT_CH), lambda i: (i, 0)),
        out_shape=jax.ShapeDtypeStruct((N, OUT_CH), jnp.float32),
    )(parts)


_GDN = jax.lax.GatherDimensionNumbers(
    offset_dims=(), collapsed_slice_dims=(0,), start_index_map=(0,))


def _bcast_lane(v, lane):
    idx = jnp.full((16,), lane, jnp.int32)
    return jax.lax.gather(
        v, idx[:, None], _GDN, (1,),
        mode=jax.lax.GatherScatterMode.PROMISE_IN_BOUNDS)


def _edge_pass_sc(tab, adst, sh2x16, epk, h_heads, split):
    c = 64
    ngrp = c // 16
    per = c // h_heads
    lane0 = 16 - h_heads
    pw = EPAD // NS if split else EPAD // NW
    nch = pw // B
    mesh = plsc.VectorSubcoreMesh(core_axis_name="c", subcore_axis_name="s",
                                  num_cores=NC, num_subcores=NS)

    def body(tab_hbm, adst_hbm, sh_hbm, epk_hbm,
             parts_hbm, shv, epkw, srcv, dstv, dgv, trows, adr, acc_sh,
             s_tab, s_ad, s_sc):
        cid = lax.axis_index("c")
        sid = lax.axis_index("s")
        wsel = sid if split else sid * NC + cid
        offs = cid * NPAD if split else 0

        zv = jnp.zeros((16,), jnp.float32)

        def zrow(i, carry):
            trows[0][i // (RT // 16), pl.ds(16 * (i % (RT // 16)), 16)] = zv
            return carry

        lax.fori_loop(0, B * (RT // 16), zrow, 0, unroll=4)
        for i in range(RPT // B):
            pltpu.sync_copy(trows[0], acc_sh.at[pl.ds(sid * RPT + i * B, B)])
        rem = RPT - (RPT // B) * B
        if rem:
            pltpu.sync_copy(trows[0].at[pl.ds(0, rem)],
                            acc_sh.at[pl.ds(sid * RPT + (RPT // B) * B, rem)])
        pltpu.sync_copy(sh_hbm, shv)
        pltpu.sync_copy(epk_hbm.at[pl.ds(wsel * pw, pw + B)], epkw)
        plsc.subcore_barrier()
        shvec = shv[cid, :]

        def edge_fn(p):
            def edge(e, carry):
                va = trows[p][e, pl.ds(c, 16)]
                vb = adr[p][e, :]
                al = va + vb
                al = jnp.where(al >= 0, al, 0.2 * al) - shvec
                ex = jnp.exp(al)
                trows[p][e, pl.ds(c, 16)] = ex
                for j in range(ngrp):
                    bc = _bcast_lane(ex, lane0 + (16 * j) // per)
                    trows[p][e, pl.ds(16 * j, 16)] = (
                        trows[p][e, pl.ds(16 * j, 16)] * bc)
                return carry
            return edge

        def start_fetch(k, q):
            base = k * B
            for g in range(B // 16):
                v = epkw[pl.ds(base + 16 * g, 16)]
                srcv[q][pl.ds(16 * g, 16)] = (
                    jnp.bitwise_and(v, 0xFFFF) + offs)
                d = jnp.right_shift(v, 16)
                dstv[q][pl.ds(16 * g, 16)] = d
                dgv[q][pl.ds(16 * g, 16)] = d + offs
            ga = pltpu.async_copy(tab_hbm.at[srcv[q]], trows[q], s_tab[q])
            gb = pltpu.async_copy(adst_hbm.at[dgv[q]], adr[q], s_ad[q])
            return ga, gb

        def step(t, carry):
            g0a, g0b = start_fetch(2 * t, 0)
            g1a, g1b = start_fetch(2 * t + 1, 1)
            g0a.wait()
            g0b.wait()
            lax.fori_loop(0, B, edge_fn(0), 0, unroll=4)
            sc0 = pltpu.async_copy(trows[0], acc_sh.at[dstv[0]], s_sc[0],
                                   add=True)
            g1a.wait()
            g1b.wait()
            lax.fori_loop(0, B, edge_fn(1), 0, unroll=4)
            sc1 = pltpu.async_copy(trows[1], acc_sh.at[dstv[1]], s_sc[1],
                                   add=True)
            sc0.wait()
            sc1.wait()
            return carry

        lax.fori_loop(0, nch // 2, step, 0)
        plsc.subcore_barrier()
        pltpu.sync_copy(acc_sh.at[pl.ds(sid * RPT, RPT)],
                        parts_hbm.at[cid, pl.ds(sid * RPT, RPT)])

    f = pl.kernel(
        body,
        out_type=jax.ShapeDtypeStruct((NC, NPAD, RT), jnp.float32),
        mesh=mesh,
        compiler_params=pltpu.CompilerParams(use_tc_tiling_on_sc=False),
        scratch_types=[
            pltpu.VMEM((2, 16), jnp.float32),
            pltpu.VMEM((pw + B,), jnp.int32),
            [pltpu.VMEM((B,), jnp.int32)] * 2,
            [pltpu.VMEM((B,), jnp.int32)] * 2,
            [pltpu.VMEM((B,), jnp.int32)] * 2,
            [pltpu.VMEM((B, RT), jnp.float32)] * 2,
            [pltpu.VMEM((B, 16), jnp.float32)] * 2,
            pltpu.VMEM_SHARED((NPAD, RT), jnp.float32),
            [pltpu.SemaphoreType.DMA] * 2,
            [pltpu.SemaphoreType.DMA] * 2,
            [pltpu.SemaphoreType.DMA] * 2,
        ],
    )
    return f(tab, adst, sh2x16, epk)


def kernel(x, edge_index, W1, att_src1, att_dst1, W2, att_src2, att_dst2):
    src = edge_index[0]
    dst = edge_index[1]
    npad_e = EPAD + B - E
    srcp = jnp.concatenate([src, jnp.zeros((npad_e,), jnp.int32)])
    dstp = jnp.concatenate([dst, jnp.full((npad_e,), N, jnp.int32)])
    epk = jnp.bitwise_or(srcp, jnp.left_shift(dstp, 16))

    eye = jnp.eye(HEADS, dtype=jnp.float32)
    As1 = (att_src1[0][:, :, None] * eye[:, None, :]).reshape(HEADS * HID, HEADS)
    Ad1 = (att_dst1[0][:, :, None] * eye[:, None, :]).reshape(HEADS * HID, HEADS)
    As2 = att_src2[0, 0][:, None]
    Ad2 = att_dst2[0, 0][:, None]

    hh = HEADS // 2
    tab1, ad1, mx1 = _project(x, W1, As1, Ad1)
    bound1 = (jnp.max(mx1[:, 0, :HEADS], axis=0)
              + jnp.max(mx1[:, 0, HEADS:], axis=0))
    shift1 = jnp.where(bound1 >= 0, bound1, 0.2 * bound1)
    padv = jnp.full((16 - hh,), PADSHIFT, jnp.float32)
    sh1 = jnp.stack([jnp.concatenate([padv, shift1[:hh]]),
                     jnp.concatenate([padv, shift1[hh:]])])
    tab1s = jnp.pad(tab1, ((0, 0), (0, NPAD - N), (0, 0))).reshape(
        2 * NPAD, RT)
    ad1s = jnp.pad(ad1, ((0, 0), (0, NPAD - N), (0, 0))).reshape(
        2 * NPAD, 16)

    parts1 = _edge_pass_sc(tab1s, ad1s, sh1, epk, hh, True)

    tab2, adst2, mx2 = _combine_project(parts1, W2, As2, Ad2)
    bound2 = jnp.max(mx2[:, 0, :1], axis=0) + jnp.max(mx2[:, 0, 1:], axis=0)
    shift2 = jnp.where(bound2 >= 0, bound2, 0.2 * bound2)
    sh2v = jnp.concatenate(
        [jnp.full((15,), PADSHIFT, jnp.float32), shift2])
    sh2 = jnp.stack([sh2v, sh2v])
    adst2p = jnp.pad(adst2, ((0, NPAD - N), (0, 0)))

    parts2 = _edge_pass_sc(tab2, adst2p, sh2, epk, 1, False)

    return _final_norm(parts2)


# speedup vs baseline: 35.4559x; 1.0035x over previous
import functools

import jax
import jax.numpy as jnp
from jax import lax
from jax.experimental import pallas as pl
from jax.experimental.pallas import tpu as pltpu
from jax.experimental.pallas import tpu_sc as plsc

N = 10000
E = 320000
IN_CH = 128
HID = 16
HEADS = 8
OUT_CH = 64

ROWS = 1000

NC = 2
NS = 16
NW = NC * NS
B = 128
EPAD = 327680
NPAD = 10112
RPT = NPAD // NS
PADSHIFT = 40.0
RT1 = 80
RT2 = 80


def _proj_kernel(x_ref, w_ref, as_ref, ad_ref,
                 tlo_ref, alo_ref, mx_ref):
    h = jnp.dot(x_ref[...], w_ref[...], preferred_element_type=jnp.float32)
    a_src = jnp.dot(h, as_ref[...], preferred_element_type=jnp.float32)
    a_dst = jnp.dot(h, ad_ref[...], preferred_element_type=jnp.float32)
    r = h.shape[0]
    hh = HEADS // 2
    z = jnp.zeros((r, 16 - hh), jnp.float32)
    tlo = jnp.concatenate([h[:, :64], z, a_src[:, :hh]], axis=-1)
    thi = jnp.concatenate([h[:, 64:], z, a_src[:, hh:]], axis=-1)
    alo = jnp.concatenate([z, a_dst[:, :hh]], axis=-1)
    ahi = jnp.concatenate([z, a_dst[:, hh:]], axis=-1)
    tlo_ref[...] = jnp.stack([tlo, thi])
    alo_ref[...] = jnp.stack([alo, ahi])
    mx = jnp.concatenate(
        [jnp.max(a_src, axis=0, keepdims=True),
         jnp.max(a_dst, axis=0, keepdims=True)], axis=-1)
    mx_ref[...] = jnp.broadcast_to(mx[None], mx_ref.shape)


def _project(x, W, As, Ad):
    c = W.shape[1]
    nh = As.shape[1]
    grid = N // ROWS
    return pl.pallas_call(
        _proj_kernel,
        grid=(grid,),
        in_specs=[
            pl.BlockSpec((ROWS, x.shape[1]), lambda i: (i, 0)),
            pl.BlockSpec((x.shape[1], c), lambda i: (0, 0)),
            pl.BlockSpec((c, nh), lambda i: (0, 0)),
            pl.BlockSpec((c, nh), lambda i: (0, 0)),
        ],
        out_specs=[
            pl.BlockSpec((2, ROWS, RT1), lambda i: (0, i, 0)),
            pl.BlockSpec((2, ROWS, 16), lambda i: (0, i, 0)),
            pl.BlockSpec((1, 8, 2 * nh), lambda i: (i, 0, 0)),
        ],
        out_shape=[
            jax.ShapeDtypeStruct((2, N, RT1), jnp.float32),
            jax.ShapeDtypeStruct((2, N, 16), jnp.float32),
            jax.ShapeDtypeStruct((grid, 8, 2 * nh), jnp.float32),
        ],
    )(x, W, As, Ad)


def _combine_proj_kernel(parts_ref, w_ref, as_ref, ad_ref,
                         tab_ref, adst_ref, mx_ref):
    p = parts_ref[...]
    acc = jnp.concatenate([p[0, :, :64], p[1, :, :64]], axis=-1)
    den = jnp.concatenate([p[0, :, 76:80], p[1, :, 76:80]], axis=-1)
    r = acc.shape[0]
    denb = jnp.repeat(den[:, :, None], HID, axis=2).reshape(r, HEADS * HID)
    v = acc / (denb + 1e-16)
    v = jnp.where(v > 0, v, jnp.exp(v) - 1.0)
    h = jnp.dot(v, w_ref[...], preferred_element_type=jnp.float32)
    a_src = jnp.dot(h, as_ref[...], preferred_element_type=jnp.float32)
    a_dst = jnp.dot(h, ad_ref[...], preferred_element_type=jnp.float32)
    z = jnp.zeros((r, 15), jnp.float32)
    tab_ref[...] = jnp.concatenate([h, z, a_src], axis=-1)
    adst_ref[...] = jnp.concatenate([z, a_dst], axis=-1)
    mx = jnp.concatenate(
        [jnp.max(a_src, axis=0, keepdims=True),
         jnp.max(a_dst, axis=0, keepdims=True)], axis=-1)
    mx_ref[...] = jnp.broadcast_to(mx[None], mx_ref.shape)


def _combine_project(parts, W, As, Ad):
    co = W.shape[1]
    ho = As.shape[1]
    grid = N // ROWS
    return pl.pallas_call(
        _combine_proj_kernel,
        grid=(grid,),
        in_specs=[
            pl.BlockSpec((2, ROWS, RT1), lambda i: (0, i, 0)),
            pl.BlockSpec((HEADS * HID, co), lambda i: (0, 0)),
            pl.BlockSpec((co, ho), lambda i: (0, 0)),
            pl.BlockSpec((co, ho), lambda i: (0, 0)),
        ],
        out_specs=[
            pl.BlockSpec((ROWS, RT2), lambda i: (i, 0)),
            pl.BlockSpec((ROWS, 16), lambda i: (i, 0)),
            pl.BlockSpec((1, 8, 2 * ho), lambda i: (i, 0, 0)),
        ],
        out_shape=[
            jax.ShapeDtypeStruct((N, RT2), jnp.float32),
            jax.ShapeDtypeStruct((N, 16), jnp.float32),
            jax.ShapeDtypeStruct((grid, 8, 2 * ho), jnp.float32),
        ],
    )(parts, W, As, Ad)


def _final_kernel(parts_ref, out_ref):
    p = parts_ref[...]
    s = p[0] + p[1]
    acc = s[:, :OUT_CH]
    den = s[:, 79:80]
    out_ref[...] = acc / (den + 1e-16)


def _final_norm(parts):
    grid = N // ROWS
    return pl.pallas_call(
        _final_kernel,
        grid=(grid,),
        in_specs=[pl.BlockSpec((2, ROWS, RT2), lambda i: (0, i, 0))],
        out_specs=pl.BlockSpec((ROWS, OUT_CH), lambda i: (i, 0)),
        out_shape=jax.ShapeDtypeStruct((N, OUT_CH), jnp.float32),
    )(parts)


_GDN = jax.lax.GatherDimensionNumbers(
    offset_dims=(), collapsed_slice_dims=(0,), start_index_map=(0,))


def _bcast_lane(v, lane):
    idx = jnp.full((16,), lane, jnp.int32)
    return jax.lax.gather(
        v, idx[:, None], _GDN, (1,),
        mode=jax.lax.GatherScatterMode.PROMISE_IN_BOUNDS)


def _edge_pass_sc(tab, adst, sh2x16, epk, h_heads, split):
    c = 64
    rt = 80
    ngrp = c // 16
    per = c // h_heads
    lane0 = 16 - h_heads
    pw = EPAD // NS if split else EPAD // NW
    nch = pw // B
    mesh = plsc.VectorSubcoreMesh(core_axis_name="c", subcore_axis_name="s",
                                  num_cores=NC, num_subcores=NS)

    def body(tab_hbm, adst_hbm, sh_hbm, epk_hbm,
             parts_hbm, shv, epkw, srcv, dstv, dgv, trows, adr, acc_sh,
             s_tab, s_ad, s_sc):
        cid = lax.axis_index("c")
        sid = lax.axis_index("s")
        wsel = sid if split else sid * NC + cid
        offs = cid * NPAD if split else 0

        zv = jnp.zeros((16,), jnp.float32)

        nz = (rt + 15) // 16

        def zrow2(e, carry):
            for g in range(nz):
                off = 16 * g if g < nz - 1 else rt - 16
                trows[0][e, pl.ds(off, 16)] = zv
            return carry

        lax.fori_loop(0, B, zrow2, 0, unroll=2)
        for i in range(RPT // B):
            pltpu.sync_copy(trows[0], acc_sh.at[pl.ds(sid * RPT + i * B, B)])
        rem = RPT - (RPT // B) * B
        if rem:
            pltpu.sync_copy(trows[0].at[pl.ds(0, rem)],
                            acc_sh.at[pl.ds(sid * RPT + (RPT // B) * B, rem)])
        pltpu.sync_copy(sh_hbm, shv)
        pltpu.sync_copy(epk_hbm.at[pl.ds(wsel * pw, pw + B)], epkw)
        plsc.subcore_barrier()
        shvec = shv[cid, :]

        def edge_fn(p):
            def edge(e, carry):
                va = trows[p][e, pl.ds(c, 16)]
                vb = adr[p][e, :]
                al = va + vb
                al = jnp.where(al >= 0, al, 0.2 * al) - shvec
                ex = jnp.exp(al)
                trows[p][e, pl.ds(c, 16)] = ex
                for j in range(ngrp):
                    bc = _bcast_lane(ex, lane0 + (16 * j) // per)
                    trows[p][e, pl.ds(16 * j, 16)] = (
                        trows[p][e, pl.ds(16 * j, 16)] * bc)
                return carry
            return edge

        def start_fetch(k, q):
            base = k * B
            for g in range(B // 16):
                v = epkw[pl.ds(base + 16 * g, 16)]
                srcv[q][pl.ds(16 * g, 16)] = (
                    jnp.bitwise_and(v, 0xFFFF) + offs)
                d = jnp.right_shift(v, 16)
                dstv[q][pl.ds(16 * g, 16)] = d
                dgv[q][pl.ds(16 * g, 16)] = d + offs
            ga = pltpu.async_copy(tab_hbm.at[srcv[q]], trows[q], s_tab[q])
            gb = pltpu.async_copy(adst_hbm.at[dgv[q]], adr[q], s_ad[q])
            return ga, gb

        def step(t, carry):
            g0a, g0b = start_fetch(2 * t, 0)
            g1a, g1b = start_fetch(2 * t + 1, 1)
            g0a.wait()
            g0b.wait()
            lax.fori_loop(0, B, edge_fn(0), 0, unroll=4)
            sc0 = pltpu.async_copy(trows[0], acc_sh.at[dstv[0]], s_sc[0],
                                   add=True)
            g1a.wait()
            g1b.wait()
            lax.fori_loop(0, B, edge_fn(1), 0, unroll=4)
            sc1 = pltpu.async_copy(trows[1], acc_sh.at[dstv[1]], s_sc[1],
                                   add=True)
            sc0.wait()
            sc1.wait()
            return carry

        lax.fori_loop(0, nch // 2, step, 0)
        plsc.subcore_barrier()
        pltpu.sync_copy(acc_sh.at[pl.ds(sid * RPT, RPT)],
                        parts_hbm.at[cid, pl.ds(sid * RPT, RPT)])

    f = pl.kernel(
        body,
        out_type=jax.ShapeDtypeStruct((NC, NPAD, rt), jnp.float32),
        mesh=mesh,
        compiler_params=pltpu.CompilerParams(use_tc_tiling_on_sc=False),
        scratch_types=[
            pltpu.VMEM((2, 16), jnp.float32),
            pltpu.VMEM((pw + B,), jnp.int32),
            [pltpu.VMEM((B,), jnp.int32)] * 2,
            [pltpu.VMEM((B,), jnp.int32)] * 2,
            [pltpu.VMEM((B,), jnp.int32)] * 2,
            [pltpu.VMEM((B, rt), jnp.float32)] * 2,
            [pltpu.VMEM((B, 16), jnp.float32)] * 2,
            pltpu.VMEM_SHARED((NPAD, rt), jnp.float32),
            [pltpu.SemaphoreType.DMA] * 2,
            [pltpu.SemaphoreType.DMA] * 2,
            [pltpu.SemaphoreType.DMA] * 2,
        ],
    )
    return f(tab, adst, sh2x16, epk)


def kernel(x, edge_index, W1, att_src1, att_dst1, W2, att_src2, att_dst2):
    src = edge_index[0]
    dst = edge_index[1]
    npad_e = EPAD + B - E
    srcp = jnp.concatenate([src, jnp.zeros((npad_e,), jnp.int32)])
    dstp = jnp.concatenate([dst, jnp.full((npad_e,), N, jnp.int32)])
    epk = jnp.bitwise_or(srcp, jnp.left_shift(dstp, 16))

    eye = jnp.eye(HEADS, dtype=jnp.float32)
    As1 = (att_src1[0][:, :, None] * eye[:, None, :]).reshape(HEADS * HID, HEADS)
    Ad1 = (att_dst1[0][:, :, None] * eye[:, None, :]).reshape(HEADS * HID, HEADS)
    As2 = att_src2[0, 0][:, None]
    Ad2 = att_dst2[0, 0][:, None]

    hh = HEADS // 2
    tab1, ad1, mx1 = _project(x, W1, As1, Ad1)
    bound1 = (jnp.max(mx1[:, 0, :HEADS], axis=0)
              + jnp.max(mx1[:, 0, HEADS:], axis=0))
    shift1 = jnp.where(bound1 >= 0, bound1, 0.2 * bound1)
    padv = jnp.full((16 - hh,), PADSHIFT, jnp.float32)
    sh1 = jnp.stack([jnp.concatenate([padv, shift1[:hh]]),
                     jnp.concatenate([padv, shift1[hh:]])])
    tab1s = jnp.pad(tab1, ((0, 0), (0, NPAD - N), (0, 0))).reshape(
        2 * NPAD, RT1)
    ad1s = jnp.pad(ad1, ((0, 0), (0, NPAD - N), (0, 0))).reshape(
        2 * NPAD, 16)

    parts1 = _edge_pass_sc(tab1s, ad1s, sh1, epk, hh, True)

    tab2, adst2, mx2 = _combine_project(parts1, W2, As2, Ad2)
    bound2 = jnp.max(mx2[:, 0, :1], axis=0) + jnp.max(mx2[:, 0, 1:], axis=0)
    shift2 = jnp.where(bound2 >= 0, bound2, 0.2 * bound2)
    sh2v = jnp.concatenate(
        [jnp.full((15,), PADSHIFT, jnp.float32), shift2])
    sh2 = jnp.stack([sh2v, sh2v])
    adst2p = jnp.pad(adst2, ((0, NPAD - N), (0, 0)))

    parts2 = _edge_pass_sc(tab2, adst2p, sh2, epk, 1, False)

    return _final_norm(parts2)


# speedup vs baseline: 35.4616x; 1.0002x over previous
import functools

import jax
import jax.numpy as jnp
from jax import lax
from jax.experimental import pallas as pl
from jax.experimental.pallas import tpu as pltpu
from jax.experimental.pallas import tpu_sc as plsc

N = 10000
E = 320000
IN_CH = 128
HID = 16
HEADS = 8
OUT_CH = 64

ROWS = 1000

NC = 2
NS = 16
NW = NC * NS
B = 128
EPAD = 327680
NPAD = 10112
RPT = NPAD // NS
PADSHIFT = 40.0
RT1 = 80
RT2 = 80


def _proj_kernel(x_ref, w_ref, as_ref, ad_ref,
                 tlo_ref, alo_ref, mx_ref):
    h = jnp.dot(x_ref[...], w_ref[...], preferred_element_type=jnp.float32)
    a_src = jnp.dot(h, as_ref[...], preferred_element_type=jnp.float32)
    a_dst = jnp.dot(h, ad_ref[...], preferred_element_type=jnp.float32)
    r = h.shape[0]
    hh = HEADS // 2
    z = jnp.zeros((r, 16 - hh), jnp.float32)
    tlo = jnp.concatenate([h[:, :64], z, a_src[:, :hh]], axis=-1)
    thi = jnp.concatenate([h[:, 64:], z, a_src[:, hh:]], axis=-1)
    alo = jnp.concatenate([z, a_dst[:, :hh]], axis=-1)
    ahi = jnp.concatenate([z, a_dst[:, hh:]], axis=-1)
    tlo_ref[...] = jnp.stack([tlo, thi])
    alo_ref[...] = jnp.stack([alo, ahi])
    mx = jnp.concatenate(
        [jnp.max(a_src, axis=0, keepdims=True),
         jnp.max(a_dst, axis=0, keepdims=True)], axis=-1)
    mx_ref[...] = jnp.broadcast_to(mx[None], mx_ref.shape)


def _project(x, W, As, Ad):
    c = W.shape[1]
    nh = As.shape[1]
    grid = N // ROWS
    return pl.pallas_call(
        _proj_kernel,
        grid=(grid,),
        in_specs=[
            pl.BlockSpec((ROWS, x.shape[1]), lambda i: (i, 0)),
            pl.BlockSpec((x.shape[1], c), lambda i: (0, 0)),
            pl.BlockSpec((c, nh), lambda i: (0, 0)),
            pl.BlockSpec((c, nh), lambda i: (0, 0)),
        ],
        out_specs=[
            pl.BlockSpec((2, ROWS, RT1), lambda i: (0, i, 0)),
            pl.BlockSpec((2, ROWS, 16), lambda i: (0, i, 0)),
            pl.BlockSpec((1, 8, 2 * nh), lambda i: (i, 0, 0)),
        ],
        out_shape=[
            jax.ShapeDtypeStruct((2, N, RT1), jnp.float32),
            jax.ShapeDtypeStruct((2, N, 16), jnp.float32),
            jax.ShapeDtypeStruct((grid, 8, 2 * nh), jnp.float32),
        ],
    )(x, W, As, Ad)


def _combine_proj_kernel(parts_ref, w_ref, as_ref, ad_ref,
                         tab_ref, adst_ref, mx_ref):
    p = parts_ref[...]
    acc = jnp.concatenate([p[0, :, :64], p[1, :, :64]], axis=-1)
    den = jnp.concatenate([p[0, :, 76:80], p[1, :, 76:80]], axis=-1)
    r = acc.shape[0]
    denb = jnp.repeat(den[:, :, None], HID, axis=2).reshape(r, HEADS * HID)
    v = acc / (denb + 1e-16)
    v = jnp.where(v > 0, v, jnp.exp(v) - 1.0)
    h = jnp.dot(v, w_ref[...], preferred_element_type=jnp.float32)
    a_src = jnp.dot(h, as_ref[...], preferred_element_type=jnp.float32)
    a_dst = jnp.dot(h, ad_ref[...], preferred_element_type=jnp.float32)
    z = jnp.zeros((r, 15), jnp.float32)
    tab_ref[...] = jnp.concatenate([h, z, a_src], axis=-1)
    adst_ref[...] = jnp.concatenate([z, a_dst], axis=-1)
    mx = jnp.concatenate(
        [jnp.max(a_src, axis=0, keepdims=True),
         jnp.max(a_dst, axis=0, keepdims=True)], axis=-1)
    mx_ref[...] = jnp.broadcast_to(mx[None], mx_ref.shape)


def _combine_project(parts, W, As, Ad):
    co = W.shape[1]
    ho = As.shape[1]
    grid = N // ROWS
    return pl.pallas_call(
        _combine_proj_kernel,
        grid=(grid,),
        in_specs=[
            pl.BlockSpec((2, ROWS, RT1), lambda i: (0, i, 0)),
            pl.BlockSpec((HEADS * HID, co), lambda i: (0, 0)),
            pl.BlockSpec((co, ho), lambda i: (0, 0)),
            pl.BlockSpec((co, ho), lambda i: (0, 0)),
        ],
        out_specs=[
            pl.BlockSpec((ROWS, RT2), lambda i: (i, 0)),
            pl.BlockSpec((ROWS, 16), lambda i: (i, 0)),
            pl.BlockSpec((1, 8, 2 * ho), lambda i: (i, 0, 0)),
        ],
        out_shape=[
            jax.ShapeDtypeStruct((N, RT2), jnp.float32),
            jax.ShapeDtypeStruct((N, 16), jnp.float32),
            jax.ShapeDtypeStruct((grid, 8, 2 * ho), jnp.float32),
        ],
    )(parts, W, As, Ad)


def _final_kernel(parts_ref, out_ref):
    p = parts_ref[...]
    s = p[0] + p[1]
    acc = s[:, :OUT_CH]
    den = s[:, 79:80]
    out_ref[...] = acc / (den + 1e-16)


def _final_norm(parts):
    grid = N // ROWS
    return pl.pallas_call(
        _final_kernel,
        grid=(grid,),
        in_specs=[pl.BlockSpec((2, ROWS, RT2), lambda i: (0, i, 0))],
        out_specs=pl.BlockSpec((ROWS, OUT_CH), lambda i: (i, 0)),
        out_shape=jax.ShapeDtypeStruct((N, OUT_CH), jnp.float32),
    )(parts)


_GDN = jax.lax.GatherDimensionNumbers(
    offset_dims=(), collapsed_slice_dims=(0,), start_index_map=(0,))


def _bcast_lane(v, lane):
    idx = jnp.full((16,), lane, jnp.int32)
    return jax.lax.gather(
        v, idx[:, None], _GDN, (1,),
        mode=jax.lax.GatherScatterMode.PROMISE_IN_BOUNDS)


def _edge_pass_sc(tab, adst, sh2x16, epk, h_heads, split):
    c = 64
    rt = 80
    ngrp = c // 16
    per = c // h_heads
    lane0 = 16 - h_heads
    pw = EPAD // NS if split else EPAD // NW
    nch = pw // B
    mesh = plsc.VectorSubcoreMesh(core_axis_name="c", subcore_axis_name="s",
                                  num_cores=NC, num_subcores=NS)

    def body(tab_hbm, adst_hbm, sh_hbm, epk_hbm,
             parts_hbm, shv, epkw, srcv, dstv, dgv, trows, adr, acc_sh,
             s_tab, s_ad, s_sc):
        cid = lax.axis_index("c")
        sid = lax.axis_index("s")
        wsel = sid if split else sid * NC + cid
        offs = cid * NPAD if split else 0

        zv = jnp.zeros((16,), jnp.float32)

        nz = (rt + 15) // 16

        def zrow2(e, carry):
            for g in range(nz):
                off = 16 * g if g < nz - 1 else rt - 16
                trows[0][e, pl.ds(off, 16)] = zv
            return carry

        lax.fori_loop(0, B, zrow2, 0, unroll=2)
        for i in range(RPT // B):
            pltpu.sync_copy(trows[0], acc_sh.at[pl.ds(sid * RPT + i * B, B)])
        rem = RPT - (RPT // B) * B
        if rem:
            pltpu.sync_copy(trows[0].at[pl.ds(0, rem)],
                            acc_sh.at[pl.ds(sid * RPT + (RPT // B) * B, rem)])
        pltpu.sync_copy(sh_hbm, shv)
        pltpu.sync_copy(epk_hbm.at[pl.ds(wsel * pw, pw + B)], epkw)
        plsc.subcore_barrier()
        shvec = shv[cid, :]

        def edge_fn(p):
            def edge(e, carry):
                va = trows[p][e, pl.ds(c, 16)]
                vb = adr[p][e, :]
                al = va + vb
                al = jnp.where(al >= 0, al, 0.2 * al) - shvec
                ex = jnp.exp(al)
                trows[p][e, pl.ds(c, 16)] = ex
                for j in range(ngrp):
                    bc = _bcast_lane(ex, lane0 + (16 * j) // per)
                    trows[p][e, pl.ds(16 * j, 16)] = (
                        trows[p][e, pl.ds(16 * j, 16)] * bc)
                return carry
            return edge

        def start_fetch(k, q):
            base = k * B
            for g in range(B // 16):
                v = epkw[pl.ds(base + 16 * g, 16)]
                srcv[q][pl.ds(16 * g, 16)] = (
                    jnp.bitwise_and(v, 0xFFFF) + offs)
                d = jnp.right_shift(v, 16)
                dstv[q][pl.ds(16 * g, 16)] = d
                dgv[q][pl.ds(16 * g, 16)] = d + offs
            ga = pltpu.async_copy(tab_hbm.at[srcv[q]], trows[q], s_tab[q])
            gb = pltpu.async_copy(adst_hbm.at[dgv[q]], adr[q], s_ad[q])
            return ga, gb

        def step(t, carry):
            g0a, g0b = start_fetch(2 * t, 0)
            g1a, g1b = start_fetch(2 * t + 1, 1)
            g0a.wait()
            g0b.wait()
            lax.fori_loop(0, B, edge_fn(0), 0, unroll=8)
            sc0 = pltpu.async_copy(trows[0], acc_sh.at[dstv[0]], s_sc[0],
                                   add=True)
            g1a.wait()
            g1b.wait()
            lax.fori_loop(0, B, edge_fn(1), 0, unroll=8)
            sc1 = pltpu.async_copy(trows[1], acc_sh.at[dstv[1]], s_sc[1],
                                   add=True)
            sc0.wait()
            sc1.wait()
            return carry

        lax.fori_loop(0, nch // 2, step, 0)
        plsc.subcore_barrier()
        pltpu.sync_copy(acc_sh.at[pl.ds(sid * RPT, RPT)],
                        parts_hbm.at[cid, pl.ds(sid * RPT, RPT)])

    f = pl.kernel(
        body,
        out_type=jax.ShapeDtypeStruct((NC, NPAD, rt), jnp.float32),
        mesh=mesh,
        compiler_params=pltpu.CompilerParams(use_tc_tiling_on_sc=False),
        scratch_types=[
            pltpu.VMEM((2, 16), jnp.float32),
            pltpu.VMEM((pw + B,), jnp.int32),
            [pltpu.VMEM((B,), jnp.int32)] * 2,
            [pltpu.VMEM((B,), jnp.int32)] * 2,
            [pltpu.VMEM((B,), jnp.int32)] * 2,
            [pltpu.VMEM((B, rt), jnp.float32)] * 2,
            [pltpu.VMEM((B, 16), jnp.float32)] * 2,
            pltpu.VMEM_SHARED((NPAD, rt), jnp.float32),
            [pltpu.SemaphoreType.DMA] * 2,
            [pltpu.SemaphoreType.DMA] * 2,
            [pltpu.SemaphoreType.DMA] * 2,
        ],
    )
    return f(tab, adst, sh2x16, epk)


def kernel(x, edge_index, W1, att_src1, att_dst1, W2, att_src2, att_dst2):
    src = edge_index[0]
    dst = edge_index[1]
    npad_e = EPAD + B - E
    srcp = jnp.concatenate([src, jnp.zeros((npad_e,), jnp.int32)])
    dstp = jnp.concatenate([dst, jnp.full((npad_e,), N, jnp.int32)])
    epk = jnp.bitwise_or(srcp, jnp.left_shift(dstp, 16))

    eye = jnp.eye(HEADS, dtype=jnp.float32)
    As1 = (att_src1[0][:, :, None] * eye[:, None, :]).reshape(HEADS * HID, HEADS)
    Ad1 = (att_dst1[0][:, :, None] * eye[:, None, :]).reshape(HEADS * HID, HEADS)
    As2 = att_src2[0, 0][:, None]
    Ad2 = att_dst2[0, 0][:, None]

    hh = HEADS // 2
    tab1, ad1, mx1 = _project(x, W1, As1, Ad1)
    bound1 = (jnp.max(mx1[:, 0, :HEADS], axis=0)
              + jnp.max(mx1[:, 0, HEADS:], axis=0))
    shift1 = jnp.where(bound1 >= 0, bound1, 0.2 * bound1)
    padv = jnp.full((16 - hh,), PADSHIFT, jnp.float32)
    sh1 = jnp.stack([jnp.concatenate([padv, shift1[:hh]]),
                     jnp.concatenate([padv, shift1[hh:]])])
    tab1s = jnp.pad(tab1, ((0, 0), (0, NPAD - N), (0, 0))).reshape(
        2 * NPAD, RT1)
    ad1s = jnp.pad(ad1, ((0, 0), (0, NPAD - N), (0, 0))).reshape(
        2 * NPAD, 16)

    parts1 = _edge_pass_sc(tab1s, ad1s, sh1, epk, hh, True)

    tab2, adst2, mx2 = _combine_project(parts1, W2, As2, Ad2)
    bound2 = jnp.max(mx2[:, 0, :1], axis=0) + jnp.max(mx2[:, 0, 1:], axis=0)
    shift2 = jnp.where(bound2 >= 0, bound2, 0.2 * bound2)
    sh2v = jnp.concatenate(
        [jnp.full((15,), PADSHIFT, jnp.float32), shift2])
    sh2 = jnp.stack([sh2v, sh2v])
    adst2p = jnp.pad(adst2, ((0, NPAD - N), (0, 0)))

    parts2 = _edge_pass_sc(tab2, adst2p, sh2, epk, 1, False)

    return _final_norm(parts2)
